# Initial kernel scaffold; baseline (speedup 1.0000x reference)
import jax
import jax.numpy as jnp
from jax.experimental import pallas as pl


def kernel(x, adj, W1, b1, W2, b2, W3, b3, L1, c1, L2, c2, L3, c3):
    raise NotImplementedError("write your pallas kernel here")


# speedup vs baseline: 1.0490x; 1.0490x over previous
import jax
import jax.numpy as jnp
from jax.experimental import pallas as pl
from jax.experimental.pallas import tpu as pltpu

N = 8192
DIM_TOUCHED = 32
NN = 256
EPS = 1e-5
BM = 256

f32 = jnp.float32
i32 = jnp.int32

_DEF = jax.lax.Precision.DEFAULT


def _dot(a, b):
    return jax.lax.dot_general(
        a, b, dimension_numbers=(((1,), (0,)), ((), ())),
        precision=_DEF, preferred_element_type=f32)


def _gcn_body(adj_ref, h_ref, w_ref, b_ref, out_ref, t_ref):
    @pl.when(pl.program_id(0) == 0)
    def _():
        t_ref[...] = _dot(h_ref[...], w_ref[...])

    y = _dot(adj_ref[...], t_ref[...])
    out_ref[...] = jnp.maximum(y + b_ref[...], 0.0)


def _gcn_layer(adj, h, W, b):
    kin, k = W.shape
    return pl.pallas_call(
        _gcn_body,
        grid=(N // BM,),
        in_specs=[
            pl.BlockSpec((BM, N), lambda i: (i, 0)),
            pl.BlockSpec((N, kin), lambda i: (0, 0)),
            pl.BlockSpec((kin, k), lambda i: (0, 0)),
            pl.BlockSpec((1, k), lambda i: (0, 0)),
        ],
        out_specs=pl.BlockSpec((BM, k), lambda i: (i, 0)),
        out_shape=jax.ShapeDtypeStruct((N, k), f32),
        scratch_shapes=[pltpu.VMEM((N, k), f32)],
    )(adj, h, W, b.reshape(1, -1))


def _bn(h):
    m = jnp.mean(h, axis=0, keepdims=True)
    v = jnp.mean((h - m) ** 2, axis=0, keepdims=True)
    return (h - m) / jnp.sqrt(v + EPS)


def _mlp_body(cat_ref, l1_ref, c1_ref, l2_ref, c2_ref, l3_ref, c3_ref, out_ref):
    m1 = _bn(jnp.maximum(_dot(cat_ref[...], l1_ref[...]) + c1_ref[...], 0.0))
    m2 = _bn(jnp.maximum(_dot(m1, l2_ref[...]) + c2_ref[...], 0.0))
    out_ref[...] = _dot(m2, l3_ref[...]) + c3_ref[...]


def _select_body(mlp_ref, out_ref):
    mlp = mlp_ref[...]
    i = jax.lax.bitcast_convert_type(mlp, i32)
    y = jnp.where(i < 0, i ^ jnp.int32(0x7FFFFFFF), i)
    k = NN + 1

    c0 = jnp.sum((y >= 0).astype(i32))
    p0 = jnp.where(c0 >= k, jnp.int32(0), jnp.int32(-2147483648))

    def body(bit, p):
        cand = p | (jnp.int32(1) << (jnp.int32(30) - bit))
        cnt = jnp.sum((y >= cand).astype(i32))
        return jnp.where(cnt >= k, cand, p)

    p = jax.lax.fori_loop(0, 31, body, p0)
    tbits = jnp.where(p < 0, p ^ jnp.int32(0x7FFFFFFF), p)
    thresh = jax.lax.bitcast_convert_type(tbits, f32)
    out_ref[...] = jnp.where(mlp > thresh, mlp * jnp.reciprocal(mlp),
                             jnp.zeros_like(mlp))


def _full_spec(*shapes):
    def mk(s):
        return pl.BlockSpec(s, lambda s=s: (0,) * len(s))
    return [mk(s) for s in shapes]


def kernel(x, adj, W1, b1, W2, b2, W3, b3, L1, c1, L2, c2, L3, c3):
    h = x[:, :DIM_TOUCHED]
    for W, b in ((W1, b1), (W2, b2), (W3, b3)):
        h = _gcn_layer(adj, h, W, b)
    cat = jnp.concatenate([h, x[:, DIM_TOUCHED:]], axis=1)

    nin, nh1 = L1.shape
    nh2 = L2.shape[1]
    mlp = pl.pallas_call(
        _mlp_body,
        in_specs=_full_spec((N, nin), (nin, nh1), (1, nh1), (nh1, nh2),
                            (1, nh2), (nh2, 1), (1, 1)),
        out_specs=pl.BlockSpec((N, 1), lambda: (0, 0)),
        out_shape=jax.ShapeDtypeStruct((N, 1), f32),
    )(cat, L1, c1.reshape(1, -1), L2, c2.reshape(1, -1), L3, c3.reshape(1, -1))

    mlp2d = mlp.reshape(64, 128)
    vac = pl.pallas_call(
        _select_body,
        in_specs=[pl.BlockSpec((64, 128), lambda: (0, 0))],
        out_specs=pl.BlockSpec((64, 128), lambda: (0, 0)),
        out_shape=jax.ShapeDtypeStruct((64, 128), f32),
    )(mlp2d)
    return vac.reshape(N, 1)
